# baseline (speedup 1.0000x reference)
import functools

import numpy as np
import jax
import jax.numpy as jnp
from jax import lax
from jax.experimental import pallas as pl
from jax.experimental.pallas import tpu as pltpu
from jax.experimental.pallas import tpu_sc as plsc

N = 10000
K = 32
D = 128

NC = 2
NS = 16
NW = NC * NS

CH = 80
NCH = N // CH

_IDX_TABLE = (
    np.arange(NCH, dtype=np.int32)[:, None, None] * (CH * K)
    + np.arange(K, dtype=np.int32)[None, :, None]
    + np.arange(CH, dtype=np.int32)[None, None, :] * K
)


def _sc_body(src_hbm, idxt_hbm, out_hbm, idx_v, acc_v, sem_idx, sem_g):
    c_id = lax.axis_index("c")
    s_id = lax.axis_index("s")
    wid = s_id * NC + c_id
    nch_w = (NCH - wid + NW - 1) // NW

    def chunk_body(i, _):
        c = wid + i * NW
        pltpu.async_copy(idxt_hbm.at[c], idx_v, sem_idx).wait()
        pltpu.async_copy(src_hbm.at[idx_v.at[0]], acc_v, sem_g).wait()

        def fire(k, _):
            pltpu.async_copy(src_hbm.at[idx_v.at[k]], acc_v, sem_g, add=True)
            return ()

        lax.fori_loop(1, K, fire, ())

        def drain(k, _):
            pltpu.make_async_copy(src_hbm.at[idx_v.at[0]], acc_v, sem_g).wait()
            return ()

        lax.fori_loop(1, K, drain, ())
        pltpu.sync_copy(acc_v, out_hbm.at[pl.ds(c * CH, CH)])
        return ()

    lax.fori_loop(0, nch_w, chunk_body, ())


_sc_segment_sum = pl.kernel(
    _sc_body,
    out_type=jax.ShapeDtypeStruct((N, D), jnp.float32),
    mesh=plsc.VectorSubcoreMesh(
        core_axis_name="c", subcore_axis_name="s", num_cores=NC, num_subcores=NS
    ),
    scratch_types=[
        pltpu.VMEM((K, CH), jnp.int32),
        pltpu.VMEM((CH, D), jnp.float32),
        pltpu.SemaphoreType.DMA,
        pltpu.SemaphoreType.DMA,
    ],
)


def _mm_body(x_ref, w_ref, o_ref):
    o_ref[...] = jnp.dot(x_ref[...], w_ref[...], preferred_element_type=jnp.float32)


def _tc_matmul(x, w):
    B = 2000
    return pl.pallas_call(
        _mm_body,
        grid=(N // B,),
        in_specs=[
            pl.BlockSpec((B, D), lambda i: (i, 0)),
            pl.BlockSpec((D, D), lambda i: (0, 0)),
        ],
        out_specs=pl.BlockSpec((B, D), lambda i: (i, 0)),
        out_shape=jax.ShapeDtypeStruct((N, D), jnp.float32),
    )(x, w)


@jax.jit
def kernel(neighbor_feature, weight):
    src = neighbor_feature.reshape(N * K, D)
    sums = _sc_segment_sum(src, jnp.asarray(_IDX_TABLE))
    return _tc_matmul(sums, weight * (1.0 / K))
